# baseline (speedup 1.0000x reference)
import functools
import math

import jax
import jax.numpy as jnp
from jax import lax
from jax.experimental import pallas as pl
from jax.experimental.pallas import tpu as pltpu
from jax.experimental.pallas import tpu_sc as plsc

D_EMBED = 64
SCALE = math.sqrt(D_EMBED)
NUM_CORES = 2
NUM_SUBCORES = 16
NUM_WORKERS = NUM_CORES * NUM_SUBCORES
LANES = 16
CHUNK = 512


def _build_sc_gather(n_rows: int):
    assert n_rows % (NUM_WORKERS * CHUNK) == 0
    rows_per_worker = n_rows // NUM_WORKERS
    n_chunks = rows_per_worker // CHUNK

    mesh = plsc.VectorSubcoreMesh(core_axis_name="c", subcore_axis_name="s")

    @functools.partial(
        pl.kernel,
        out_type=jax.ShapeDtypeStruct((n_rows, D_EMBED), jnp.float32),
        mesh=mesh,
        scratch_types=[
            pltpu.VMEM((CHUNK,), jnp.int32),
            pltpu.VMEM((CHUNK, D_EMBED), jnp.float32),
            pltpu.SemaphoreType.DMA,
        ],
        compiler_params=pltpu.CompilerParams(use_tc_tiling_on_sc=False),
    )
    def sc_gather(idx_hbm, tab_hbm, out_hbm, idx_v, rows_v, sem):
        wid = lax.axis_index("s") * NUM_CORES + lax.axis_index("c")
        base = wid * rows_per_worker

        @pl.loop(0, n_chunks)
        def _chunk(ci):
            off = base + ci * CHUNK
            pltpu.sync_copy(idx_hbm.at[pl.ds(off, CHUNK)], idx_v)
            pltpu.async_copy(tab_hbm.at[idx_v], rows_v, sem).wait()

            @pl.loop(0, CHUNK)
            def _row(r):
                for j in range(D_EMBED // LANES):
                    sl = pl.ds(j * LANES, LANES)
                    rows_v[r, sl] = rows_v[r, sl] * SCALE

            pltpu.sync_copy(rows_v, out_hbm.at[pl.ds(off, CHUNK)])

    return sc_gather


def kernel(x, table):
    b, t = x.shape
    idx = x.reshape(-1).astype(jnp.int32)
    out = _build_sc_gather(idx.shape[0])(idx, table)
    return out.reshape(b, t, D_EMBED)


# speedup vs baseline: 1.1403x; 1.1403x over previous
import functools
import math

import jax
import jax.numpy as jnp
from jax import lax
from jax.experimental import pallas as pl
from jax.experimental.pallas import tpu as pltpu
from jax.experimental.pallas import tpu_sc as plsc

D_EMBED = 64
SCALE = math.sqrt(D_EMBED)
NUM_CORES = 2
NUM_SUBCORES = 16
NUM_WORKERS = NUM_CORES * NUM_SUBCORES
LANES = 16
CHUNK = 512
NBUF = 2
ROW_UNROLL = 4


def _build_sc_gather(n_rows: int):
    assert n_rows % (NUM_WORKERS * CHUNK * NBUF) == 0
    rows_per_worker = n_rows // NUM_WORKERS
    n_chunks = rows_per_worker // CHUNK

    mesh = plsc.VectorSubcoreMesh(core_axis_name="c", subcore_axis_name="s")

    @functools.partial(
        pl.kernel,
        out_type=jax.ShapeDtypeStruct((n_rows, D_EMBED), jnp.float32),
        mesh=mesh,
        scratch_types=[
            pltpu.VMEM((rows_per_worker,), jnp.int32),
            pltpu.VMEM((NBUF, CHUNK, D_EMBED), jnp.float32),
            pltpu.SemaphoreType.DMA((NBUF,)),
            pltpu.SemaphoreType.DMA((NBUF,)),
        ],
        compiler_params=pltpu.CompilerParams(use_tc_tiling_on_sc=False),
    )
    def sc_gather(idx_hbm, tab_hbm, out_hbm, idx_v, rows_v, gsem, ssem):
        wid = lax.axis_index("s") * NUM_CORES + lax.axis_index("c")
        base = wid * rows_per_worker
        pltpu.sync_copy(idx_hbm.at[pl.ds(base, rows_per_worker)], idx_v)

        def gather(ci, b):
            return pltpu.make_async_copy(
                tab_hbm.at[idx_v.at[pl.ds(ci * CHUNK, CHUNK)]],
                rows_v.at[b],
                gsem.at[b],
            )

        def store(ci, b):
            return pltpu.make_async_copy(
                rows_v.at[b],
                out_hbm.at[pl.ds(base + ci * CHUNK, CHUNK)],
                ssem.at[b],
            )

        def scale(b):
            @pl.loop(0, CHUNK, step=ROW_UNROLL)
            def _rows(r):
                for dr in range(ROW_UNROLL):
                    for j in range(D_EMBED // LANES):
                        sl = pl.ds(j * LANES, LANES)
                        rows_v[b, r + dr, sl] = rows_v[b, r + dr, sl] * SCALE

        gather(0, 0).start()

        @pl.loop(0, n_chunks // NBUF)
        def _group(g):
            ci0 = g * NBUF
            for b in range(NBUF):
                ci = ci0 + b
                nb = (b + 1) % NBUF

                @pl.when(ci + 1 < n_chunks)
                def _start_next():
                    @pl.when(ci >= NBUF - 1)
                    def _drain_nb():
                        store(0, nb).wait()

                    gather(ci + 1, nb).start()

                gather(ci, b).wait()
                scale(b)
                store(ci, b).start()

        for b in range(NBUF):
            store(0, b).wait()

    return sc_gather


def kernel(x, table):
    b, t = x.shape
    idx = x.reshape(-1).astype(jnp.int32)
    out = _build_sc_gather(idx.shape[0])(idx, table)
    return out.reshape(b, t, D_EMBED)
